# baseline (speedup 1.0000x reference)
import jax
import jax.numpy as jnp
from jax import lax
from jax.experimental import pallas as pl
from jax.experimental.pallas import tpu as pltpu
from jax.experimental.pallas import tpu_sc as plsc

B = 16384
D = 32
N_CORES = 2
N_SUBCORES = 16
NW = N_CORES * N_SUBCORES
BPW = B // NW
CHUNK = 128
NCHUNK = BPW // CHUNK
HALF = BPW // 2
LANES = 16


def _body(user_hbm, arm_hbm, ut_hbm, at_hbm, out_hbm,
          idx_u, idx_a, div_u, div_a, flat_u, flat_a,
          rows_u, rows_a, out_v, sem):
    wid = lax.axis_index("s") * N_CORES + lax.axis_index("c")
    rbase = wid * NCHUNK

    pltpu.sync_copy(user_hbm.at[pl.ds(rbase, NCHUNK)], idx_u)
    pltpu.sync_copy(arm_hbm.at[pl.ds(rbase, NCHUNK)], idx_a)

    for j in range(NCHUNK):
        for k in range(CHUNK // LANES):
            s = pl.ds(k * LANES, LANES)
            vu = idx_u[j, s]
            va = idx_a[j, s]
            div_u[j, s] = lax.shift_right_logical(vu, 2)
            div_a[j, s] = lax.shift_right_logical(va, 2)
            fs = pl.ds(j * CHUNK + k * LANES, LANES)
            flat_u[fs] = vu
            flat_a[fs] = va

    def compute_half(h):
        def group(g, carry):
            gg = h * (HALF // LANES) + g
            iu = flat_u[pl.ds(gg * LANES, LANES)]
            ia = flat_a[pl.ds(gg * LANES, LANES)]
            base_u = lax.shift_left(lax.bitwise_and(iu, 3), 5)
            base_a = lax.shift_left(lax.bitwise_and(ia, 3), 5)
            rows = g * LANES + lax.iota(jnp.int32, LANES)
            acc = jnp.zeros((LANES,), jnp.float32)
            for d in range(D):
                cu = plsc.load_gather(rows_u, [rows, base_u + d])
                ca = plsc.load_gather(rows_a, [rows, base_a + d])
                acc = acc + cu * ca
            out_v[pl.ds(gg * LANES, LANES)] = acc
            return carry
        lax.fori_loop(0, HALF // LANES, group, jnp.int32(0))

    for h in range(2):
        handles = []
        for j in range(2):
            c = h * 2 + j
            handles.append(pltpu.async_copy(
                ut_hbm.at[div_u.at[c]], rows_u.at[pl.ds(j * CHUNK, CHUNK)],
                sem))
            handles.append(pltpu.async_copy(
                at_hbm.at[div_a.at[c]], rows_a.at[pl.ds(j * CHUNK, CHUNK)],
                sem))
        for hd in handles:
            hd.wait()
        compute_half(h)

    pltpu.sync_copy(out_v, out_hbm.at[pl.ds(wid * BPW, BPW)])


@jax.jit
def kernel(user, arm, user_table, arm_table):
    user2d = user.astype(jnp.int32).reshape(B // CHUNK, CHUNK)
    arm2d = arm.astype(jnp.int32).reshape(B // CHUNK, CHUNK)
    ut_wide = user_table.reshape(-1, 128)
    at_wide = arm_table.reshape(-1, 128)
    mesh = plsc.VectorSubcoreMesh(core_axis_name="c", subcore_axis_name="s",
                                  num_cores=N_CORES, num_subcores=N_SUBCORES)
    f = pl.kernel(
        _body,
        out_type=jax.ShapeDtypeStruct((B,), jnp.float32),
        mesh=mesh,
        scratch_types=[
            pltpu.VMEM((NCHUNK, CHUNK), jnp.int32),
            pltpu.VMEM((NCHUNK, CHUNK), jnp.int32),
            pltpu.VMEM((NCHUNK, CHUNK), jnp.int32),
            pltpu.VMEM((NCHUNK, CHUNK), jnp.int32),
            pltpu.VMEM((BPW,), jnp.int32),
            pltpu.VMEM((BPW,), jnp.int32),
            pltpu.VMEM((HALF, 128), jnp.float32),
            pltpu.VMEM((HALF, 128), jnp.float32),
            pltpu.VMEM((BPW,), jnp.float32),
            pltpu.SemaphoreType.DMA,
        ],
        compiler_params=pltpu.CompilerParams(needs_layout_passes=False),
    )
    return f(user2d, arm2d, ut_wide, at_wide)


# speedup vs baseline: 2.6989x; 2.6989x over previous
import jax
import jax.numpy as jnp
from jax import lax
from jax.experimental import pallas as pl
from jax.experimental.pallas import tpu as pltpu
from jax.experimental.pallas import tpu_sc as plsc

B = 16384
D = 32
N_CORES = 2
N_SUBCORES = 16
BPT = B // N_SUBCORES
HALF_D = D // N_CORES
NG = BPT // 16
ACHUNKS = BPT // 128


def _body(user_hbm, arm_hbm, ut_hbm, aw_hbm, out_hbm,
          idx_u, idx_a, div_a, win, arow, aT, out_v, s0, s1, s2):
    cid = lax.axis_index("c")
    sid = lax.axis_index("s")
    base = sid * BPT
    dbase = pl.multiple_of(cid * HALF_D, HALF_D)

    pltpu.sync_copy(user_hbm.at[pl.ds(base, BPT)], idx_u)
    pltpu.sync_copy(arm_hbm.at[pl.ds(base, BPT)], idx_a)

    def mkdiv(i, carry):
        s = pl.ds(i * 16, 16)
        div_a[s] = lax.shift_right_logical(idx_a[s], 2)
        return carry
    lax.fori_loop(0, BPT // 16, mkdiv, jnp.int32(0))

    for c in range(ACHUNKS):
        pltpu.async_copy(aw_hbm.at[div_a.at[pl.ds(c * 128, 128)]],
                         arow, s2).wait()

        def aext(gg, carry):
            j = gg * 16 + lax.iota(jnp.int32, 16)
            va = idx_a[pl.ds(c * 128 + gg * 16, 16)]
            cb = lax.shift_left(lax.bitwise_and(va, 3), 5) + dbase
            for dl in range(HALF_D):
                v = plsc.load_gather(arow, [j, cb + dl])
                aT[pl.ds(dl * BPT + c * 128 + gg * 16, 16)] = v
            return carry
        lax.fori_loop(0, 8, aext, jnp.int32(0))

    lanes = lax.iota(jnp.int32, 16)

    def issue(g, b):
        vu = idx_u[pl.ds(g * 16, 16)]
        colv = lax.bitwise_and(vu, ~127)
        sem = s0 if b == 0 else s1
        for k in range(16):
            c0 = pl.multiple_of(colv[k], 128)
            pltpu.async_copy(
                ut_hbm.at[pl.ds(dbase, HALF_D), pl.ds(c0, 128)],
                win.at[pl.ds(b * 256 + k * 16, HALF_D), :], sem)

    issue(jnp.int32(0), 0)
    issue(jnp.int32(1), 1)

    def group_body(g, b):
        sem = s0 if b == 0 else s1
        for k in range(16):
            pltpu.make_async_copy(
                ut_hbm.at[pl.ds(dbase, HALF_D), pl.ds(0, 128)],
                win.at[pl.ds(b * 256 + k * 16, HALF_D), :], sem).wait()
        vu = idx_u[pl.ds(g * 16, 16)]
        cols = lax.bitwise_and(vu, 127)
        rows0 = b * 256 + lanes * 16
        acc = jnp.zeros((16,), jnp.float32)
        for dl in range(HALF_D):
            u = plsc.load_gather(win, [rows0 + dl, cols])
            a = aT[pl.ds(dl * BPT + g * 16, 16)]
            acc = acc + u * a
        out_v[pl.ds(g * 16, 16)] = acc

        @pl.when(g + 2 < NG)
        def _():
            issue(g + 2, b)

    def loop(g, carry):
        @pl.when(lax.bitwise_and(g, 1) == 0)
        def _():
            group_body(g, 0)

        @pl.when(lax.bitwise_and(g, 1) == 1)
        def _():
            group_body(g, 1)
        return carry
    lax.fori_loop(0, NG, loop, jnp.int32(0))

    pltpu.sync_copy(out_v, out_hbm.at[pl.ds(cid * B + base, BPT)])


@jax.jit
def kernel(user, arm, user_table, arm_table):
    ut_t = user_table.T
    aw = arm_table.reshape(-1, 128)
    mesh = plsc.VectorSubcoreMesh(core_axis_name="c", subcore_axis_name="s",
                                  num_cores=N_CORES, num_subcores=N_SUBCORES)
    f = pl.kernel(
        _body,
        out_type=jax.ShapeDtypeStruct((2 * B,), jnp.float32),
        mesh=mesh,
        scratch_types=[
            pltpu.VMEM((BPT,), jnp.int32),
            pltpu.VMEM((BPT,), jnp.int32),
            pltpu.VMEM((BPT,), jnp.int32),
            pltpu.VMEM((512, 128), jnp.float32),
            pltpu.VMEM((128, 128), jnp.float32),
            pltpu.VMEM((HALF_D * BPT,), jnp.float32),
            pltpu.VMEM((BPT,), jnp.float32),
            pltpu.SemaphoreType.DMA,
            pltpu.SemaphoreType.DMA,
            pltpu.SemaphoreType.DMA,
        ],
        compiler_params=pltpu.CompilerParams(needs_layout_passes=False),
    )
    parts = f(user.astype(jnp.int32), arm.astype(jnp.int32), ut_t, aw)
    return parts[:B] + parts[B:]


# speedup vs baseline: 3.1014x; 1.1491x over previous
import jax
import jax.numpy as jnp
from jax import lax
from jax.experimental import pallas as pl
from jax.experimental.pallas import tpu as pltpu
from jax.experimental.pallas import tpu_sc as plsc

B = 16384
D = 32
N_CORES = 2
N_SUBCORES = 16
BPT = B // N_SUBCORES
HALF_D = D // N_CORES
NG = BPT // 16
DEPTH = 3


def _body(user_hbm, arm_hbm, ut_hbm, aw_hbm, out_hbm,
          idx_u, idx_a, div_a, win, armw, out_v,
          su0, su1, su2, sa0, sa1, sa2):
    cid = lax.axis_index("c")
    sid = lax.axis_index("s")
    base = sid * BPT
    dbase = pl.multiple_of(cid * HALF_D, HALF_D)
    sus = (su0, su1, su2)
    sas = (sa0, sa1, sa2)

    pltpu.sync_copy(user_hbm.at[pl.ds(base, BPT)], idx_u)
    pltpu.sync_copy(arm_hbm.at[pl.ds(base, BPT)], idx_a)

    def mkdiv(i, carry):
        s = pl.ds(i * 16, 16)
        div_a[s] = lax.shift_right_logical(idx_a[s], 2)
        return carry
    lax.fori_loop(0, BPT // 16, mkdiv, jnp.int32(0))

    lanes = lax.iota(jnp.int32, 16)

    def issue(g, b):
        vu = idx_u[pl.ds(g * 16, 16)]
        colv = lax.bitwise_and(vu, ~127)
        for k in range(16):
            c0 = pl.multiple_of(colv[k], 128)
            pltpu.async_copy(
                ut_hbm.at[pl.ds(dbase, HALF_D), pl.ds(c0, 128)],
                win.at[pl.ds(b * 256 + k * 16, HALF_D), :], sus[b])
        pltpu.async_copy(aw_hbm.at[div_a.at[pl.ds(g * 16, 16)]],
                         armw.at[pl.ds(b * 16, 16), :], sas[b])

    for b in range(DEPTH):
        issue(jnp.int32(b), b)

    def group_body(g, b):
        for k in range(16):
            pltpu.make_async_copy(
                ut_hbm.at[pl.ds(dbase, HALF_D), pl.ds(0, 128)],
                win.at[pl.ds(b * 256 + k * 16, HALF_D), :], sus[b]).wait()
        pltpu.make_async_copy(aw_hbm.at[div_a.at[pl.ds(g * 16, 16)]],
                              armw.at[pl.ds(b * 16, 16), :], sas[b]).wait()
        vu = idx_u[pl.ds(g * 16, 16)]
        va = idx_a[pl.ds(g * 16, 16)]
        cols = lax.bitwise_and(vu, 127)
        cb = lax.shift_left(lax.bitwise_and(va, 3), 5) + dbase
        rows0 = b * 256 + lanes * 16
        arows = b * 16 + lanes
        acc = jnp.zeros((16,), jnp.float32)
        for dl in range(HALF_D):
            u = plsc.load_gather(win, [rows0 + dl, cols])
            a = plsc.load_gather(armw, [arows, cb + dl])
            acc = acc + u * a
        out_v[pl.ds(g * 16, 16)] = acc

        @pl.when(g + DEPTH < NG)
        def _():
            issue(g + DEPTH, b)

    def loop(g, carry):
        m = lax.rem(g, DEPTH)
        for b in range(DEPTH):
            @pl.when(m == b)
            def _(b=b):
                group_body(g, b)
        return carry
    lax.fori_loop(0, NG, loop, jnp.int32(0))

    pltpu.sync_copy(out_v, out_hbm.at[pl.ds(cid * B + base, BPT)])


@jax.jit
def kernel(user, arm, user_table, arm_table):
    ut_t = user_table.T
    aw = arm_table.reshape(-1, 128)
    mesh = plsc.VectorSubcoreMesh(core_axis_name="c", subcore_axis_name="s",
                                  num_cores=N_CORES, num_subcores=N_SUBCORES)
    f = pl.kernel(
        _body,
        out_type=jax.ShapeDtypeStruct((2 * B,), jnp.float32),
        mesh=mesh,
        scratch_types=[
            pltpu.VMEM((BPT,), jnp.int32),
            pltpu.VMEM((BPT,), jnp.int32),
            pltpu.VMEM((BPT,), jnp.int32),
            pltpu.VMEM((DEPTH * 256, 128), jnp.float32),
            pltpu.VMEM((DEPTH * 16, 128), jnp.float32),
            pltpu.VMEM((BPT,), jnp.float32),
            pltpu.SemaphoreType.DMA,
            pltpu.SemaphoreType.DMA,
            pltpu.SemaphoreType.DMA,
            pltpu.SemaphoreType.DMA,
            pltpu.SemaphoreType.DMA,
            pltpu.SemaphoreType.DMA,
        ],
        compiler_params=pltpu.CompilerParams(needs_layout_passes=False),
    )
    parts = f(user.astype(jnp.int32), arm.astype(jnp.int32), ut_t, aw)
    return parts[:B] + parts[B:]
